# baseline (speedup 1.0000x reference)
import functools

import jax
import jax.numpy as jnp
from jax import lax
from jax.experimental import pallas as pl
from jax.experimental.pallas import tpu as pltpu
from jax.experimental.pallas import tpu_sc as plsc

NC = 2
NS = 16
NW = NC * NS
LANE = 128
BLK_A = 128
BLK_B = 128


def _body(V, D, B, H, wT, idxT, wtail, out, ltab,
          a0, a1, lb0, lb1, a_tl, ix0, ix1, g0, g1, o0, o1,
          sa0, sa1, sl0, sl1, si0, si1, sg0, sg1, so0, so1, bsem):
    dw = D // 2
    wid = lax.axis_index("s") * NC + lax.axis_index("c")

    iotas = [lax.iota(jnp.int32, 16) + 16 * g for g in range(LANE // 16)]
    cols = [jnp.full((16,), k, jnp.int32) for k in range(dw)]

    a_bufs = (a0, a1)
    lb_bufs = (lb0, lb1)
    sa = (sa0, sa1)
    sl = (sl0, sl1)

    nblk = V // BLK_A
    tail = V - nblk * BLK_A
    per = nblk // NW
    na = per + 1 + (per + 1) % 2

    def blk_of(i):
        return (wid * per + i) % nblk

    def a_in(i, u):
        return pltpu.make_async_copy(
            wT.at[:, pl.ds(blk_of(i) * BLK_A, BLK_A)], a_bufs[u], sa[u])

    def a_out(i, u):
        return pltpu.make_async_copy(
            lb_bufs[u], ltab.at[pl.ds(blk_of(i) * BLK_A, BLK_A)], sl[u])

    def transpose_a(u):
        src = a_bufs[u].bitcast(jnp.int32)
        dst = lb_bufs[u]
        for k in range(dw):
            for g in range(BLK_A // 16):
                plsc.store_scatter(dst, [iotas[g], cols[k]],
                                   src[k, pl.ds(16 * g, 16)])

    a_in(0, 0).start()

    def phase_a(j, _):
        for u in (0, 1):
            i = 2 * j + u

            @pl.when(i + 1 < na)
            def _():
                a_in(i + 1, 1 - u).start()

            a_in(i, u).wait()

            @pl.when(i >= 2)
            def _():
                a_out(i - 2, u).wait()

            transpose_a(u)
            a_out(i, u).start()
        return 0

    lax.fori_loop(0, na // 2, phase_a, 0)
    a_out(na - 2, 0).wait()
    a_out(na - 1, 1).wait()

    @pl.when(wid == NW - 1)
    def _tail():
        pltpu.async_copy(wtail, a_tl, sa0).wait()
        t32 = a_tl.bitcast(jnp.int32)
        for k in range(dw):
            for g in range(tail // 16):
                plsc.store_scatter(lb0, [iotas[g], cols[k]],
                                   t32[k, pl.ds(16 * g, 16)])
        pltpu.async_copy(lb0.at[pl.ds(0, tail)],
                         ltab.at[pl.ds(nblk * BLK_A, tail)], sl0).wait()

    for tc in range(NC):
        for ts in range(NS):
            pl.semaphore_signal(bsem, 1, device_id={"c": tc, "s": ts})
    pl.semaphore_wait(bsem, NW)

    ntb = B // BLK_B
    nb = (H * ntb) // NW
    ix = (ix0, ix1)
    gb = (g0, g1)
    ob = (o0, o1)
    si = (si0, si1)
    sg = (sg0, sg1)
    so = (so0, so1)

    def b_idx_in(i, u):
        fb = wid * nb + i
        return pltpu.make_async_copy(
            idxT.at[fb // ntb, pl.ds((fb % ntb) * BLK_B, BLK_B)], ix[u], si[u])

    def b_gather(u):
        return pltpu.make_async_copy(ltab.at[ix[u]], gb[u], sg[u])

    def b_out(i, u):
        fb = wid * nb + i
        return pltpu.make_async_copy(
            ob[u], out.at[fb // ntb, :, pl.ds((fb % ntb) * BLK_B, BLK_B)],
            so[u])

    def transpose_b(u):
        src = gb[u]
        dst = ob[u].bitcast(jnp.int32)
        for k in range(dw):
            for g in range(BLK_B // 16):
                dst[k, pl.ds(16 * g, 16)] = plsc.load_gather(
                    src, [iotas[g], cols[k]])

    b_idx_in(0, 0).start()
    b_idx_in(0, 0).wait()
    b_gather(0).start()
    b_idx_in(1, 1).start()

    def phase_b(j, _):
        for u in (0, 1):
            i = 2 * j + u

            @pl.when(i + 1 < nb)
            def _():
                b_idx_in(i + 1, 1 - u).wait()
                b_gather(1 - u).start()

            b_gather(u).wait()

            @pl.when(i + 2 < nb)
            def _():
                b_idx_in(i + 2, u).start()

            @pl.when(i >= 2)
            def _():
                b_out(i - 2, u).wait()

            transpose_b(u)
            b_out(i, u).start()
        return 0

    lax.fori_loop(0, nb // 2, phase_b, 0)
    b_out(nb - 2, 0).wait()
    b_out(nb - 1, 1).wait()


def kernel(indices, weight):
    B, H = indices.shape
    V, D = weight.shape
    dw = D // 2
    assert B % LANE == 0 and (H * (B // LANE)) % NW == 0

    nblk = V // BLK_A
    tail = V - nblk * BLK_A
    wT = weight.T
    wtail = lax.slice(wT, (0, nblk * BLK_A), (D, V))

    mesh = plsc.VectorSubcoreMesh(core_axis_name="c", subcore_axis_name="s")
    body = functools.partial(_body, V, D, B, H)
    out = pl.kernel(
        body,
        out_type=jax.ShapeDtypeStruct((H, D, B), weight.dtype),
        mesh=mesh,
        scratch_types=[
            pltpu.HBM((V, dw), jnp.int32),
            pltpu.VMEM((D, BLK_A), weight.dtype),
            pltpu.VMEM((D, BLK_A), weight.dtype),
            pltpu.VMEM((LANE, dw), jnp.int32),
            pltpu.VMEM((LANE, dw), jnp.int32),
            pltpu.VMEM((D, tail), weight.dtype),
            pltpu.VMEM((BLK_B,), jnp.int32),
            pltpu.VMEM((BLK_B,), jnp.int32),
            pltpu.VMEM((BLK_B, dw), jnp.int32),
            pltpu.VMEM((BLK_B, dw), jnp.int32),
            pltpu.VMEM((D, BLK_B), weight.dtype),
            pltpu.VMEM((D, BLK_B), weight.dtype),
            pltpu.SemaphoreType.DMA,
            pltpu.SemaphoreType.DMA,
            pltpu.SemaphoreType.DMA,
            pltpu.SemaphoreType.DMA,
            pltpu.SemaphoreType.DMA,
            pltpu.SemaphoreType.DMA,
            pltpu.SemaphoreType.DMA,
            pltpu.SemaphoreType.DMA,
            pltpu.SemaphoreType.DMA,
            pltpu.SemaphoreType.DMA,
            pltpu.SemaphoreType.REGULAR,
        ],
        compiler_params=pltpu.CompilerParams(use_tc_tiling_on_sc=True,
                                             needs_layout_passes=False),
    )(wT, indices.T, wtail)
    return out.transpose(2, 0, 1)
